# baseline (speedup 1.0000x reference)
import functools

import jax
import jax.numpy as jnp
from jax import lax
from jax.experimental import pallas as pl
from jax.experimental.pallas import tpu as pltpu
from jax.experimental.pallas import tpu_sc as plsc

CAP = 300
CPAD = 304
CHUNK = 512


def _route_kernel(E, G, x_ref, wr_ref, br_ref, dst_ref, carry_ref):
    j = pl.program_id(0)

    @pl.when(j == 0)
    def _():
        carry_ref[...] = jnp.zeros_like(carry_ref)

    x = x_ref[0]
    r = jnp.dot(x, wr_ref[...], preferred_element_type=jnp.float32)
    rt = r.T + br_ref[...]
    m = jnp.max(rt, axis=0, keepdims=True)
    ex = jnp.exp(rt - m)
    p = ex / jnp.sum(ex, axis=0, keepdims=True)
    eidx = lax.broadcasted_iota(jnp.int32, p.shape, 0)
    pm = jnp.max(p, axis=0, keepdims=True)
    ids = jnp.min(jnp.where(p == pm, eidx, E), axis=0, keepdims=True)
    oh = (eidx == ids).astype(jnp.float32)
    ri = lax.broadcasted_iota(jnp.int32, (CHUNK, CHUNK), 0)
    ci = lax.broadcasted_iota(jnp.int32, (CHUNK, CHUNK), 1)
    triu = (ri <= ci).astype(jnp.float32)
    csum = jnp.dot(oh, triu, preferred_element_type=jnp.float32)
    total = carry_ref[...] + csum
    carry_ref[...] = total[:, CHUNK - 1 : CHUNK]
    posf = jnp.sum(total * oh, axis=0, keepdims=True) - 1.0
    pos = posf.astype(jnp.int32)
    keep = pos < CAP
    nvec = j * CHUNK + lax.broadcasted_iota(jnp.int32, (1, CHUNK), 1)
    dst = jnp.where(keep, ids * CPAD + pos, G + nvec)
    dst_ref[...] = dst.reshape(1, 1, CHUNK)


def _route(inp, Wr, br):
    B, S, D = inp.shape
    N = B * S
    E = br.shape[0]
    G = E * CPAD
    dst = pl.pallas_call(
        functools.partial(_route_kernel, E, G),
        grid=(N // CHUNK,),
        in_specs=[
            pl.BlockSpec((1, CHUNK, D), lambda j: (0, j, 0)),
            pl.BlockSpec((D, E), lambda j: (0, 0)),
            pl.BlockSpec((E, 1), lambda j: (0, 0)),
        ],
        out_specs=pl.BlockSpec((1, 1, CHUNK), lambda j: (j, 0, 0)),
        out_shape=jax.ShapeDtypeStruct((N // CHUNK, 1, CHUNK), jnp.int32),
        scratch_shapes=[pltpu.VMEM((E, 1), jnp.float32)],
    )(inp, Wr, br.reshape(E, 1))
    return dst.reshape(N)


def _dispatch(x, dst, W_ROWS):
    B, S, D = x.shape
    N = B * S
    info = plsc.get_sparse_core_info()
    NC, NS = info.num_cores, info.num_subcores
    NW = NC * NS
    tpw = N // NW
    mesh = plsc.VectorSubcoreMesh(core_axis_name="c", subcore_axis_name="s")

    hpw = tpw // 2

    @functools.partial(
        pl.kernel,
        mesh=mesh,
        out_type=jax.ShapeDtypeStruct((W_ROWS, D), jnp.float32),
        scratch_types=[
            pltpu.VMEM((hpw,), jnp.int32),
            pltpu.VMEM((hpw,), jnp.int32),
            pltpu.VMEM((hpw, D), jnp.float32),
            pltpu.VMEM((hpw, D), jnp.float32),
            pltpu.SemaphoreType.DMA,
            pltpu.SemaphoreType.DMA,
            pltpu.SemaphoreType.DMA,
            pltpu.SemaphoreType.DMA,
        ],
    )
    def k2(x_hbm, dst_hbm, w_hbm, d0, d1, b0, b1, sl0, sl1, ss0, ss1):
        wid = lax.axis_index("s") * NC + lax.axis_index("c")
        base = wid * tpw
        ld0 = pltpu.async_copy(x_hbm.at[0, pl.ds(base, hpw)], b0, sl0)
        ld1 = pltpu.async_copy(x_hbm.at[0, pl.ds(base + hpw, hpw)], b1, sl1)
        pltpu.sync_copy(dst_hbm.at[pl.ds(base, hpw)], d0)
        pltpu.sync_copy(dst_hbm.at[pl.ds(base + hpw, hpw)], d1)
        ld0.wait()
        st0 = pltpu.async_copy(b0, w_hbm.at[d0], ss0)
        ld1.wait()
        st1 = pltpu.async_copy(b1, w_hbm.at[d1], ss1)
        st0.wait()
        st1.wait()

    return k2(x, dst)


def _ffn_kernel(w_ref, we_ref, be_ref, y_ref):
    xb = w_ref[...].astype(jnp.bfloat16)
    wb = we_ref[0].astype(jnp.bfloat16)
    y = jnp.dot(xb, wb, preferred_element_type=jnp.float32)
    y_ref[...] = y + be_ref[0]


def _ffn(w, We, be):
    R, D = w.shape
    E = be.shape[0]
    hb = CPAD // 2
    ybig = pl.pallas_call(
        _ffn_kernel,
        grid=(2 * E,),
        in_specs=[
            pl.BlockSpec((hb, D), lambda i: (i, 0)),
            pl.BlockSpec((1, D, D), lambda i: (i // 2, 0, 0)),
            pl.BlockSpec((1, 1, D), lambda i: (i // 2, 0, 0)),
        ],
        out_specs=pl.BlockSpec((hb, D), lambda i: (i, 0)),
        out_shape=jax.ShapeDtypeStruct((R, D), jnp.float32),
        input_output_aliases={0: 0},
    )(w, We, be.reshape(E, 1, D))
    return ybig


def _combine(ybig, dst, N, D):
    info = plsc.get_sparse_core_info()
    NC, NS = info.num_cores, info.num_subcores
    NW = NC * NS
    tpw = N // NW
    mesh = plsc.VectorSubcoreMesh(core_axis_name="c", subcore_axis_name="s")

    hpw = tpw // 2

    @functools.partial(
        pl.kernel,
        mesh=mesh,
        out_type=jax.ShapeDtypeStruct((N, D), jnp.float32),
        scratch_types=[
            pltpu.VMEM((hpw,), jnp.int32),
            pltpu.VMEM((hpw,), jnp.int32),
            pltpu.VMEM((hpw, D), jnp.float32),
            pltpu.VMEM((hpw, D), jnp.float32),
            pltpu.SemaphoreType.DMA,
            pltpu.SemaphoreType.DMA,
            pltpu.SemaphoreType.DMA,
            pltpu.SemaphoreType.DMA,
        ],
    )
    def k4(ybig_hbm, dst_hbm, out_hbm, d0, d1, b0, b1, sg0, sg1, so0, so1):
        wid = lax.axis_index("s") * NC + lax.axis_index("c")
        base = wid * tpw
        pltpu.sync_copy(dst_hbm.at[pl.ds(base, hpw)], d0)
        pltpu.sync_copy(dst_hbm.at[pl.ds(base + hpw, hpw)], d1)
        g0 = pltpu.async_copy(ybig_hbm.at[d0], b0, sg0)
        g1 = pltpu.async_copy(ybig_hbm.at[d1], b1, sg1)
        g0.wait()
        st0 = pltpu.async_copy(b0, out_hbm.at[pl.ds(base, hpw)], so0)
        g1.wait()
        st1 = pltpu.async_copy(b1, out_hbm.at[pl.ds(base + hpw, hpw)], so1)
        st0.wait()
        st1.wait()

    return k4(ybig, dst)


def kernel(input, Wr, br, We, be):
    B, S, D = input.shape
    E = br.shape[0]
    N = B * S
    G = E * CPAD

    dst = _route(input, Wr, br)
    w = _dispatch(input, dst, G + N)
    ybig = _ffn(w, We, be)
    out = _combine(ybig, dst, N, D)
    return out.reshape(B, S, D)


# speedup vs baseline: 1.1501x; 1.1501x over previous
import functools

import jax
import jax.numpy as jnp
from jax import lax
from jax.experimental import pallas as pl
from jax.experimental.pallas import tpu as pltpu
from jax.experimental.pallas import tpu_sc as plsc

CAP = 300
CPAD = 304
CHUNK = 512


def _route_kernel(E, G, x_ref, wr_ref, br_ref, dst_ref, carry_ref):
    j = pl.program_id(0)

    @pl.when(j == 0)
    def _():
        carry_ref[...] = jnp.zeros_like(carry_ref)

    x = x_ref[0]
    r = jnp.dot(x, wr_ref[...], preferred_element_type=jnp.float32)
    rt = r.T + br_ref[...]
    m = jnp.max(rt, axis=0, keepdims=True)
    ex = jnp.exp(rt - m)
    p = ex / jnp.sum(ex, axis=0, keepdims=True)
    eidx = lax.broadcasted_iota(jnp.int32, p.shape, 0)
    pm = jnp.max(p, axis=0, keepdims=True)
    ids = jnp.min(jnp.where(p == pm, eidx, E), axis=0, keepdims=True)
    oh = (eidx == ids).astype(jnp.float32)
    ri = lax.broadcasted_iota(jnp.int32, (CHUNK, CHUNK), 0)
    ci = lax.broadcasted_iota(jnp.int32, (CHUNK, CHUNK), 1)
    triu = (ri <= ci).astype(jnp.float32)
    csum = jnp.dot(oh, triu, preferred_element_type=jnp.float32)
    total = carry_ref[...] + csum
    carry_ref[...] = total[:, CHUNK - 1 : CHUNK]
    posf = jnp.sum(total * oh, axis=0, keepdims=True) - 1.0
    pos = posf.astype(jnp.int32)
    keep = pos < CAP
    nvec = j * CHUNK + lax.broadcasted_iota(jnp.int32, (1, CHUNK), 1)
    dst = jnp.where(keep, ids * CPAD + pos, G + nvec)
    dst_ref[...] = dst.reshape(1, 1, CHUNK)


def _route(inp, Wr, br):
    B, S, D = inp.shape
    N = B * S
    E = br.shape[0]
    G = E * CPAD
    dst = pl.pallas_call(
        functools.partial(_route_kernel, E, G),
        grid=(N // CHUNK,),
        in_specs=[
            pl.BlockSpec((1, CHUNK, D), lambda j: (0, j, 0)),
            pl.BlockSpec((D, E), lambda j: (0, 0)),
            pl.BlockSpec((E, 1), lambda j: (0, 0)),
        ],
        out_specs=pl.BlockSpec((1, 1, CHUNK), lambda j: (j, 0, 0)),
        out_shape=jax.ShapeDtypeStruct((N // CHUNK, 1, CHUNK), jnp.int32),
        scratch_shapes=[pltpu.VMEM((E, 1), jnp.float32)],
    )(inp, Wr, br.reshape(E, 1))
    return dst.reshape(N)


def _dispatch(x, dst, W_ROWS):
    B, S, D = x.shape
    N = B * S
    info = plsc.get_sparse_core_info()
    NC, NS = info.num_cores, info.num_subcores
    NW = NC * NS
    tpw = N // NW
    mesh = plsc.VectorSubcoreMesh(core_axis_name="c", subcore_axis_name="s")

    hpw = tpw // 2

    @functools.partial(
        pl.kernel,
        mesh=mesh,
        out_type=jax.ShapeDtypeStruct((W_ROWS, D), jnp.float32),
        scratch_types=[
            pltpu.VMEM((hpw,), jnp.int32),
            pltpu.VMEM((hpw,), jnp.int32),
            pltpu.VMEM((hpw, D), jnp.float32),
            pltpu.VMEM((hpw, D), jnp.float32),
            pltpu.SemaphoreType.DMA,
            pltpu.SemaphoreType.DMA,
            pltpu.SemaphoreType.DMA,
            pltpu.SemaphoreType.DMA,
        ],
    )
    def k2(x_hbm, dst_hbm, w_hbm, d0, d1, b0, b1, sl0, sl1, ss0, ss1):
        wid = lax.axis_index("s") * NC + lax.axis_index("c")
        base = wid * tpw
        ld0 = pltpu.async_copy(x_hbm.at[0, pl.ds(base, hpw)], b0, sl0)
        ld1 = pltpu.async_copy(x_hbm.at[0, pl.ds(base + hpw, hpw)], b1, sl1)
        pltpu.sync_copy(dst_hbm.at[pl.ds(base, hpw)], d0)
        pltpu.sync_copy(dst_hbm.at[pl.ds(base + hpw, hpw)], d1)
        ld0.wait()
        st0 = pltpu.async_copy(b0, w_hbm.at[d0], ss0)
        ld1.wait()
        st1 = pltpu.async_copy(b1, w_hbm.at[d1], ss1)
        st0.wait()
        st1.wait()

    return k2(x, dst)


def _ffn_kernel(w_ref, we_ref, be_ref, y_ref):
    xb = w_ref[...].astype(jnp.bfloat16)
    wb = we_ref[0].astype(jnp.bfloat16)
    y = jnp.dot(xb, wb, preferred_element_type=jnp.float32)
    y_ref[...] = y + be_ref[0]


def _ffn(w, We, be):
    R, D = w.shape
    E = be.shape[0]
    ybig = pl.pallas_call(
        _ffn_kernel,
        grid=(E,),
        in_specs=[
            pl.BlockSpec((CPAD, D), lambda i: (i, 0)),
            pl.BlockSpec((1, D, D), lambda i: (i, 0, 0)),
            pl.BlockSpec((1, 1, D), lambda i: (i, 0, 0)),
        ],
        out_specs=pl.BlockSpec((CPAD, D), lambda i: (i, 0)),
        out_shape=jax.ShapeDtypeStruct((R, D), jnp.float32),
        input_output_aliases={0: 0},
    )(w, We, be.reshape(E, 1, D))
    return ybig


def _combine(ybig, dst, N, D):
    info = plsc.get_sparse_core_info()
    NC, NS = info.num_cores, info.num_subcores
    NW = NC * NS
    tpw = N // NW
    mesh = plsc.VectorSubcoreMesh(core_axis_name="c", subcore_axis_name="s")

    hpw = tpw // 2

    @functools.partial(
        pl.kernel,
        mesh=mesh,
        out_type=jax.ShapeDtypeStruct((N, D), jnp.float32),
        scratch_types=[
            pltpu.VMEM((hpw,), jnp.int32),
            pltpu.VMEM((hpw,), jnp.int32),
            pltpu.VMEM((hpw, D), jnp.float32),
            pltpu.VMEM((hpw, D), jnp.float32),
            pltpu.SemaphoreType.DMA,
            pltpu.SemaphoreType.DMA,
            pltpu.SemaphoreType.DMA,
            pltpu.SemaphoreType.DMA,
        ],
    )
    def k4(ybig_hbm, dst_hbm, out_hbm, d0, d1, b0, b1, sg0, sg1, so0, so1):
        wid = lax.axis_index("s") * NC + lax.axis_index("c")
        base = wid * tpw
        pltpu.sync_copy(dst_hbm.at[pl.ds(base, hpw)], d0)
        pltpu.sync_copy(dst_hbm.at[pl.ds(base + hpw, hpw)], d1)
        g0 = pltpu.async_copy(ybig_hbm.at[d0], b0, sg0)
        g1 = pltpu.async_copy(ybig_hbm.at[d1], b1, sg1)
        g0.wait()
        st0 = pltpu.async_copy(b0, out_hbm.at[pl.ds(base, hpw)], so0)
        g1.wait()
        st1 = pltpu.async_copy(b1, out_hbm.at[pl.ds(base + hpw, hpw)], so1)
        st0.wait()
        st1.wait()

    return k4(ybig, dst)


def kernel(input, Wr, br, We, be):
    B, S, D = input.shape
    E = br.shape[0]
    N = B * S
    G = E * CPAD

    dst = _route(input, Wr, br)
    w = _dispatch(input, dst, G + N)
    ybig = _ffn(w, We, be)
    out = _combine(ybig, dst, N, D)
    return out.reshape(B, S, D)
